# baseline (speedup 1.0000x reference)
import functools

import jax
import jax.numpy as jnp
from jax.experimental import pallas as pl
from jax.experimental.pallas import tpu as pltpu

_NUM_CLASSES = 400
_N_TILES = 8


def _head_kernel(xs_ref, xf_ref, wct_ref, bc_ref, ws_ref, wf_ref, bp_ref,
                 out_ref, acc_pool_ref, acc_cls_ref, *,
                 cb_s, cb_f, inv_s, inv_f, n_tiles):
    j = pl.program_id(1)

    @pl.when(j == 0)
    def _init():
        acc_pool_ref[...] = jnp.zeros_like(acc_pool_ref)
        acc_cls_ref[...] = jnp.zeros_like(acc_cls_ref)

    sums_s = jnp.sum(xs_ref[0], axis=1, keepdims=True) * inv_s
    wct = wct_ref[pl.ds(j * cb_s, cb_s), :]
    acc_pool_ref[...] += jax.lax.dot_general(
        sums_s, wct, (((0,), (0,)), ((), ())),
        preferred_element_type=jnp.float32)

    sums_f = jnp.sum(xf_ref[0], axis=1, keepdims=True) * inv_f
    wf = wf_ref[pl.ds(j * cb_f, cb_f), :]
    acc_cls_ref[...] += jax.lax.dot_general(
        sums_f, wf, (((0,), (0,)), ((), ())),
        preferred_element_type=jnp.float32)

    @pl.when(j == n_tiles - 1)
    def _finalize():
        pooled = acc_pool_ref[...] + bc_ref[...]
        cls = jnp.dot(pooled, ws_ref[...],
                      preferred_element_type=jnp.float32)
        out_ref[0] = cls + acc_cls_ref[...] + bp_ref[...]


def _head_forward(x_slow, x_fast, wct, bc, ws, wf, bp, *, n_tiles):
    b, cs, t, h, w = x_slow.shape
    _, cf, tf, hf, wf_sp = x_fast.shape
    thw_s = t * h * w
    thw_f = tf * hf * wf_sp
    assert cs % n_tiles == 0 and cf % n_tiles == 0
    cb_s = cs // n_tiles
    cb_f = cf // n_tiles
    dim_inner = wct.shape[1]
    nc_pad = bp.shape[-1]

    xs = x_slow.reshape(b, cs, thw_s)
    xf = x_fast.reshape(b, cf, thw_f)

    kernel_fn = functools.partial(
        _head_kernel, cb_s=cb_s, cb_f=cb_f,
        inv_s=1.0 / thw_s, inv_f=1.0 / thw_f, n_tiles=n_tiles)

    out = pl.pallas_call(
        kernel_fn,
        out_shape=jax.ShapeDtypeStruct((b, 1, nc_pad), jnp.float32),
        grid=(b, n_tiles),
        in_specs=[
            pl.BlockSpec((1, cb_s, thw_s), lambda i, j: (i, j, 0)),
            pl.BlockSpec((1, cb_f, thw_f), lambda i, j: (i, j, 0)),
            pl.BlockSpec((cs, dim_inner), lambda i, j: (0, 0)),
            pl.BlockSpec((1, dim_inner), lambda i, j: (0, 0)),
            pl.BlockSpec((dim_inner, nc_pad), lambda i, j: (0, 0)),
            pl.BlockSpec((cf, nc_pad), lambda i, j: (0, 0)),
            pl.BlockSpec((1, nc_pad), lambda i, j: (0, 0)),
        ],
        out_specs=pl.BlockSpec((1, 1, nc_pad), lambda i, j: (i, 0, 0)),
        scratch_shapes=[pltpu.VMEM((1, dim_inner), jnp.float32),
                        pltpu.VMEM((1, nc_pad), jnp.float32)],
        compiler_params=pltpu.CompilerParams(
            dimension_semantics=("parallel", "arbitrary"),
            vmem_limit_bytes=48 * 1024 * 1024),
    )(xs, xf, wct, bc, ws, wf, bp)
    return out[:, 0, :]


def kernel(x_slow, x_fast, conv5_wt, conv5_b, proj_w_slow, proj_w_fast, proj_b):
    out = _head_forward(x_slow, x_fast, conv5_wt, conv5_b,
                        proj_w_slow, proj_w_fast, proj_b, n_tiles=_N_TILES)
    return out[:, :_NUM_CLASSES]


# speedup vs baseline: 5.8099x; 5.8099x over previous
import functools

import jax
import jax.numpy as jnp
from jax.experimental import pallas as pl
from jax.experimental.pallas import tpu as pltpu

_NUM_CLASSES = 400
_N_TILES = 4


def _head_kernel(xs_ref, xf_ref, wct_ref, bc_ref, ws_ref, wf_ref, bp_ref,
                 out_ref, acc_s_ref, acc_f_ref, *, inv_s, inv_f, n_tiles):
    j = pl.program_id(1)

    @pl.when(j == 0)
    def _init():
        acc_s_ref[...] = jnp.zeros_like(acc_s_ref)
        acc_f_ref[...] = jnp.zeros_like(acc_f_ref)

    acc_s_ref[...] += jnp.sum(xs_ref[0], axis=0, keepdims=True)
    acc_f_ref[...] += jnp.sum(xf_ref[0], axis=0, keepdims=True)

    @pl.when(j == n_tiles - 1)
    def _finalize():
        pooled = acc_s_ref[...] * inv_s
        pooled = jnp.dot(pooled, wct_ref[...],
                         preferred_element_type=jnp.float32) + bc_ref[...]
        cls = jnp.dot(pooled, ws_ref[...],
                      preferred_element_type=jnp.float32)
        cls += jnp.dot(acc_f_ref[...] * inv_f, wf_ref[...],
                       preferred_element_type=jnp.float32)
        out_ref[0] = cls + bp_ref[...]


def _head_forward(x_slow, x_fast, wct, bc, ws, wf, bp, *, n_tiles):
    b, cs, t, h, w = x_slow.shape
    _, cf, tf, hf, wf_sp = x_fast.shape
    thw_s = t * h * w
    thw_f = tf * hf * wf_sp

    xs = x_slow.transpose(0, 3, 4, 2, 1).reshape(b, thw_s, cs)
    xf = x_fast.transpose(0, 3, 4, 2, 1).reshape(b, thw_f, cf)

    assert thw_s % n_tiles == 0 and thw_f % n_tiles == 0
    ts = thw_s // n_tiles
    tf_blk = thw_f // n_tiles
    assert ts % 8 == 0 and tf_blk % 8 == 0

    dim_inner = wct.shape[1]
    nc_pad = bp.shape[-1]

    kernel_fn = functools.partial(
        _head_kernel, inv_s=1.0 / thw_s, inv_f=1.0 / thw_f, n_tiles=n_tiles)

    out = pl.pallas_call(
        kernel_fn,
        out_shape=jax.ShapeDtypeStruct((b, 1, nc_pad), jnp.float32),
        grid=(b, n_tiles),
        in_specs=[
            pl.BlockSpec((1, ts, cs), lambda i, j: (i, j, 0)),
            pl.BlockSpec((1, tf_blk, cf), lambda i, j: (i, j, 0)),
            pl.BlockSpec((cs, dim_inner), lambda i, j: (0, 0)),
            pl.BlockSpec((1, dim_inner), lambda i, j: (0, 0)),
            pl.BlockSpec((dim_inner, nc_pad), lambda i, j: (0, 0)),
            pl.BlockSpec((cf, nc_pad), lambda i, j: (0, 0)),
            pl.BlockSpec((1, nc_pad), lambda i, j: (0, 0)),
        ],
        out_specs=pl.BlockSpec((1, 1, nc_pad), lambda i, j: (i, 0, 0)),
        scratch_shapes=[pltpu.VMEM((1, cs), jnp.float32),
                        pltpu.VMEM((1, cf), jnp.float32)],
        compiler_params=pltpu.CompilerParams(
            dimension_semantics=("parallel", "arbitrary"),
            vmem_limit_bytes=48 * 1024 * 1024),
    )(xs, xf, wct, bc, ws, wf, bp)
    return out[:, 0, :]


def kernel(x_slow, x_fast, conv5_wt, conv5_b, proj_w_slow, proj_w_fast, proj_b):
    out = _head_forward(x_slow, x_fast, conv5_wt, conv5_b,
                        proj_w_slow, proj_w_fast, proj_b, n_tiles=_N_TILES)
    return out[:, :_NUM_CLASSES]


# speedup vs baseline: 6.7241x; 1.1574x over previous
import functools

import jax
import jax.numpy as jnp
from jax.experimental import pallas as pl
from jax.experimental.pallas import tpu as pltpu

_NUM_CLASSES = 400
_N_TILES = 2


def _head_kernel(xs_ref, xf_ref, wct_ref, bc_ref, ws_ref, wf_ref, bp_ref,
                 out_ref, acc_s_ref, acc_f_ref, *, inv_s, inv_f, n_tiles):
    j = pl.program_id(1)

    @pl.when(j == 0)
    def _init():
        acc_s_ref[...] = jnp.zeros_like(acc_s_ref)
        acc_f_ref[...] = jnp.zeros_like(acc_f_ref)

    acc_s_ref[...] += jnp.sum(xs_ref[0], axis=0, keepdims=True)
    acc_f_ref[...] += jnp.sum(xf_ref[0], axis=0, keepdims=True)

    @pl.when(j == n_tiles - 1)
    def _finalize():
        pooled = acc_s_ref[...] * inv_s
        pooled = jnp.dot(pooled, wct_ref[...],
                         preferred_element_type=jnp.float32) + bc_ref[...]
        cls = jnp.dot(pooled, ws_ref[...],
                      preferred_element_type=jnp.float32)
        cls += jnp.dot(acc_f_ref[...] * inv_f, wf_ref[...],
                       preferred_element_type=jnp.float32)
        out_ref[0] = cls + bp_ref[...]


def _head_forward(x_slow, x_fast, wct, bc, ws, wf, bp, *, n_tiles):
    b, cs, t, h, w = x_slow.shape
    _, cf, tf, hf, wf_sp = x_fast.shape
    thw_s = t * h * w
    thw_f = tf * hf * wf_sp

    xs = x_slow.transpose(0, 3, 4, 2, 1).reshape(b, thw_s, cs)
    xf = x_fast.transpose(0, 3, 4, 2, 1).reshape(b, thw_f, cf)

    assert thw_s % n_tiles == 0 and thw_f % n_tiles == 0
    ts = thw_s // n_tiles
    tf_blk = thw_f // n_tiles
    assert ts % 8 == 0 and tf_blk % 8 == 0

    dim_inner = wct.shape[1]
    nc_pad = bp.shape[-1]

    kernel_fn = functools.partial(
        _head_kernel, inv_s=1.0 / thw_s, inv_f=1.0 / thw_f, n_tiles=n_tiles)

    out = pl.pallas_call(
        kernel_fn,
        out_shape=jax.ShapeDtypeStruct((b, 1, nc_pad), jnp.float32),
        grid=(b, n_tiles),
        in_specs=[
            pl.BlockSpec((1, ts, cs), lambda i, j: (i, j, 0)),
            pl.BlockSpec((1, tf_blk, cf), lambda i, j: (i, j, 0)),
            pl.BlockSpec((cs, dim_inner), lambda i, j: (0, 0)),
            pl.BlockSpec((1, dim_inner), lambda i, j: (0, 0)),
            pl.BlockSpec((dim_inner, nc_pad), lambda i, j: (0, 0)),
            pl.BlockSpec((cf, nc_pad), lambda i, j: (0, 0)),
            pl.BlockSpec((1, nc_pad), lambda i, j: (0, 0)),
        ],
        out_specs=pl.BlockSpec((1, 1, nc_pad), lambda i, j: (i, 0, 0)),
        scratch_shapes=[pltpu.VMEM((1, cs), jnp.float32),
                        pltpu.VMEM((1, cf), jnp.float32)],
        compiler_params=pltpu.CompilerParams(
            dimension_semantics=("parallel", "arbitrary"),
            vmem_limit_bytes=48 * 1024 * 1024),
    )(xs, xf, wct, bc, ws, wf, bp)
    return out[:, 0, :]


def kernel(x_slow, x_fast, conv5_wt, conv5_b, proj_w_slow, proj_w_fast, proj_b):
    out = _head_forward(x_slow, x_fast, conv5_wt, conv5_b,
                        proj_w_slow, proj_w_fast, proj_b, n_tiles=_N_TILES)
    return out[:, :_NUM_CLASSES]


# speedup vs baseline: 7.0834x; 1.0534x over previous
import functools

import jax
import jax.numpy as jnp
from jax.experimental import pallas as pl
from jax.experimental.pallas import tpu as pltpu

_NUM_CLASSES = 400
_N_TILES = 1


def _head_kernel(xs_ref, xf_ref, wct_ref, bc_ref, ws_ref, wf_ref, bp_ref,
                 out_ref, acc_s_ref, acc_f_ref, *, inv_s, inv_f, n_tiles):
    j = pl.program_id(1)

    @pl.when(j == 0)
    def _init():
        acc_s_ref[...] = jnp.zeros_like(acc_s_ref)
        acc_f_ref[...] = jnp.zeros_like(acc_f_ref)

    acc_s_ref[...] += jnp.sum(xs_ref[0], axis=0, keepdims=True)
    acc_f_ref[...] += jnp.sum(xf_ref[0], axis=0, keepdims=True)

    @pl.when(j == n_tiles - 1)
    def _finalize():
        pooled = acc_s_ref[...] * inv_s
        pooled = jnp.dot(pooled, wct_ref[...],
                         preferred_element_type=jnp.float32) + bc_ref[...]
        cls = jnp.dot(pooled, ws_ref[...],
                      preferred_element_type=jnp.float32)
        cls += jnp.dot(acc_f_ref[...] * inv_f, wf_ref[...],
                       preferred_element_type=jnp.float32)
        out_ref[0] = cls + bp_ref[...]


def _head_forward(x_slow, x_fast, wct, bc, ws, wf, bp, *, n_tiles):
    b, cs, t, h, w = x_slow.shape
    _, cf, tf, hf, wf_sp = x_fast.shape
    thw_s = t * h * w
    thw_f = tf * hf * wf_sp

    xs = x_slow.transpose(0, 3, 4, 2, 1).reshape(b, thw_s, cs)
    xf = x_fast.transpose(0, 3, 4, 2, 1).reshape(b, thw_f, cf)

    assert thw_s % n_tiles == 0 and thw_f % n_tiles == 0
    ts = thw_s // n_tiles
    tf_blk = thw_f // n_tiles
    assert ts % 8 == 0 and tf_blk % 8 == 0

    dim_inner = wct.shape[1]
    nc_pad = bp.shape[-1]

    kernel_fn = functools.partial(
        _head_kernel, inv_s=1.0 / thw_s, inv_f=1.0 / thw_f, n_tiles=n_tiles)

    out = pl.pallas_call(
        kernel_fn,
        out_shape=jax.ShapeDtypeStruct((b, 1, nc_pad), jnp.float32),
        grid=(b, n_tiles),
        in_specs=[
            pl.BlockSpec((1, ts, cs), lambda i, j: (i, j, 0)),
            pl.BlockSpec((1, tf_blk, cf), lambda i, j: (i, j, 0)),
            pl.BlockSpec((cs, dim_inner), lambda i, j: (0, 0)),
            pl.BlockSpec((1, dim_inner), lambda i, j: (0, 0)),
            pl.BlockSpec((dim_inner, nc_pad), lambda i, j: (0, 0)),
            pl.BlockSpec((cf, nc_pad), lambda i, j: (0, 0)),
            pl.BlockSpec((1, nc_pad), lambda i, j: (0, 0)),
        ],
        out_specs=pl.BlockSpec((1, 1, nc_pad), lambda i, j: (i, 0, 0)),
        scratch_shapes=[pltpu.VMEM((1, cs), jnp.float32),
                        pltpu.VMEM((1, cf), jnp.float32)],
        compiler_params=pltpu.CompilerParams(
            dimension_semantics=("parallel", "arbitrary"),
            vmem_limit_bytes=63 * 1024 * 1024),
    )(xs, xf, wct, bc, ws, wf, bp)
    return out[:, 0, :]


def kernel(x_slow, x_fast, conv5_wt, conv5_b, proj_w_slow, proj_w_fast, proj_b):
    out = _head_forward(x_slow, x_fast, conv5_wt, conv5_b,
                        proj_w_slow, proj_w_fast, proj_b, n_tiles=_N_TILES)
    return out[:, :_NUM_CLASSES]
